# baseline (speedup 1.0000x reference)
import functools

import jax
import jax.numpy as jnp
from jax import lax
from jax.experimental import pallas as pl
from jax.experimental.pallas import tpu as pltpu
from jax.experimental.pallas import tpu_sc as plsc

_NC, _NS = 2, 16
_NW = _NC * _NS


def _pack_body(tbT_ref, tcT_ref, wb_ref, pbp_ref, cp_ref):
    B = tbT_ref.shape[0]
    Ec = tcT_ref.shape[1]
    LB = tbT_ref.shape[2]
    eye = (lax.broadcasted_iota(jnp.int32, (Ec, Ec), 0)
           == lax.broadcasted_iota(jnp.int32, (Ec, Ec), 1)).astype(jnp.float32)
    dn = (((0,), (1,)), ((), ()))
    ccols = []
    pcols = []
    for b in range(B):
        ccols.append(lax.dot_general(tcT_ref[b], eye, dn,
                                     preferred_element_type=jnp.float32))
        pcols.append(lax.dot_general(tbT_ref[b], wb_ref[...], dn,
                                     preferred_element_type=jnp.float32))
    cp_ref[...] = jnp.concatenate(ccols, axis=1)
    pbp_ref[...] = jnp.concatenate(
        pcols + [jnp.zeros((LB, 128 - B), jnp.float32)], axis=1)


def _tc_pack(mat_bT, mat_cT, w_b):
    B, Eb, L1 = mat_bT.shape
    Ec = mat_cT.shape[1]
    LB = 2048
    return pl.pallas_call(
        _pack_body,
        grid=(pl.cdiv(L1, LB),),
        in_specs=[
            pl.BlockSpec((B, Eb, LB), lambda i: (0, 0, i)),
            pl.BlockSpec((B, Ec, LB), lambda i: (0, 0, i)),
            pl.BlockSpec((1, Eb), lambda i: (0, 0)),
        ],
        out_specs=[
            pl.BlockSpec((LB, 128), lambda i: (i, 0)),
            pl.BlockSpec((LB, B * Ec), lambda i: (i, 0)),
        ],
        out_shape=[
            jax.ShapeDtypeStruct((L1, 128), jnp.float32),
            jax.ShapeDtypeStruct((L1, B * Ec), jnp.float32),
        ],
    )(mat_bT, mat_cT, w_b.reshape(1, Eb))


def _sc_gather(pb_pack, c_pack, idx_b, idx_c, L):
    rows_w = L // _NW
    mesh = plsc.VectorSubcoreMesh(core_axis_name="c", subcore_axis_name="s")

    @functools.partial(
        pl.kernel,
        out_type=(jax.ShapeDtypeStruct((L, 128), jnp.float32),
                  jax.ShapeDtypeStruct((L, 128), jnp.float32)),
        mesh=mesh,
        scratch_types=[
            pltpu.VMEM((_NW, rows_w), jnp.int32),
            pltpu.VMEM((_NW, rows_w), jnp.int32),
            pltpu.VMEM((rows_w, 128), jnp.float32),
            pltpu.VMEM((rows_w, 128), jnp.float32),
            pltpu.SemaphoreType.DMA,
        ],
    )
    def gather_k(pbp_h, cp_h, ib_h, ic_h, pbg_h, cg_h,
                 ib_v, ic_v, rb_v, rc_v, sem):
        wid = lax.axis_index("s") * _NC + lax.axis_index("c")
        base = wid * rows_w
        pltpu.sync_copy(ib_h, ib_v)
        pltpu.sync_copy(ic_h, ic_v)
        cb = pltpu.async_copy(pbp_h.at[ib_v.at[wid]], rb_v, sem)
        cc = pltpu.async_copy(cp_h.at[ic_v.at[wid]], rc_v, sem)
        cb.wait()
        cc.wait()
        pltpu.sync_copy(rb_v, pbg_h.at[pl.ds(base, rows_w)])
        pltpu.sync_copy(rc_v, cg_h.at[pl.ds(base, rows_w)])

    return gather_k(pb_pack, c_pack, idx_b, idx_c)


def _dense_body(pbg_ref, cg_ref, dtT_ref, wc_ref, l1b_ref, a_ref,
                bias_ref, l2_ref, l2b_ref, out_ref):
    B = dtT_ref.shape[1]
    Ec = wc_ref.shape[0]
    for b in range(B):
        c_b = cg_ref[:, b * Ec:(b + 1) * Ec]
        rc = jnp.dot(c_b, wc_ref[...], preferred_element_type=jnp.float32)
        x = jnp.dot(c_b, l1b_ref[...], preferred_element_type=jnp.float32)
        x = x + dtT_ref[:, b:b + 1] * a_ref[...] + bias_ref[...]
        xa = 1.0 / (1.0 + jnp.exp(-x))
        t = jnp.dot(xa, l2_ref[...], preferred_element_type=jnp.float32)
        rate = pbg_ref[:, b:b + 1] + rc + t + l2b_ref[...]
        out_ref[:, b:b + 1] = (jnp.maximum(rate, 0.0)
                               + jnp.log1p(jnp.exp(-jnp.abs(rate))))


def kernel(mat_b, mat_c, arr_b_idx, arr_c_idx, arr_delta_t,
           w_b, w_c, lin1a_w, lin1a_b, lin1b_w, lin1b_b, lin2_w, lin2_b):
    B, L1, Eb = mat_b.shape
    _, L2, Ec = mat_c.shape
    L = arr_b_idx.shape[1]
    rows_w = L // _NW

    ib = arr_b_idx.reshape(_NW, rows_w).astype(jnp.int32)
    ic = arr_c_idx.reshape(_NW, rows_w).astype(jnp.int32)

    pb_pack, c_pack = _tc_pack(jnp.transpose(mat_b, (0, 2, 1)),
                               jnp.transpose(mat_c, (0, 2, 1)), w_b)

    pbg, cg = _sc_gather(pb_pack, c_pack, ib, ic, L)

    dtT = arr_delta_t.astype(jnp.float32).T
    wc_col = w_c.reshape(Ec, 1)
    l1bT = lin1b_w.T
    a_row = lin1a_w.reshape(1, -1)
    bias_row = (lin1a_b + lin1b_b).reshape(1, -1)
    l2_col = lin2_w.reshape(-1, 1)
    l2b = lin2_b.reshape(1, 1)

    H = lin1b_w.shape[0]
    out = pl.pallas_call(
        _dense_body,
        in_specs=[
            pl.BlockSpec((L, 128), lambda: (0, 0)),
            pl.BlockSpec((L, 128), lambda: (0, 0)),
            pl.BlockSpec((L, B), lambda: (0, 0)),
            pl.BlockSpec((Ec, 1), lambda: (0, 0)),
            pl.BlockSpec((Ec, H), lambda: (0, 0)),
            pl.BlockSpec((1, H), lambda: (0, 0)),
            pl.BlockSpec((1, H), lambda: (0, 0)),
            pl.BlockSpec((H, 1), lambda: (0, 0)),
            pl.BlockSpec((1, 1), lambda: (0, 0)),
        ],
        out_specs=pl.BlockSpec((L, B), lambda: (0, 0)),
        out_shape=jax.ShapeDtypeStruct((L, B), jnp.float32),
    )(pbg, cg, dtT, wc_col, l1bT, a_row, bias_row, l2_col, l2b)

    return out.T


# speedup vs baseline: 1.0171x; 1.0171x over previous
import functools

import jax
import jax.numpy as jnp
from jax import lax
from jax.experimental import pallas as pl
from jax.experimental.pallas import tpu as pltpu
from jax.experimental.pallas import tpu_sc as plsc

_NC, _NS = 2, 16
_NW = _NC * _NS


def _pack_body(tbT_ref, tcT_ref, wb_ref, pbp_ref, cp_ref):
    B = tbT_ref.shape[0]
    Ec = tcT_ref.shape[1]
    LB = tbT_ref.shape[2]
    eye = (lax.broadcasted_iota(jnp.int32, (Ec, Ec), 0)
           == lax.broadcasted_iota(jnp.int32, (Ec, Ec), 1)).astype(jnp.float32)
    del LB
    dn = (((0,), (1,)), ((), ()))
    for b in range(B):
        cp_ref[:, b * Ec:(b + 1) * Ec] = lax.dot_general(
            tcT_ref[b], eye, dn, preferred_element_type=jnp.float32)
        pbp_ref[:, b:b + 1] = lax.dot_general(
            tbT_ref[b], wb_ref[...], dn, preferred_element_type=jnp.float32)


def _tc_pack(mat_bT, mat_cT, w_b):
    B, Eb, L1 = mat_bT.shape
    Ec = mat_cT.shape[1]
    LB = 2048
    return pl.pallas_call(
        _pack_body,
        grid=(pl.cdiv(L1, LB),),
        in_specs=[
            pl.BlockSpec((B, Eb, LB), lambda i: (0, 0, i)),
            pl.BlockSpec((B, Ec, LB), lambda i: (0, 0, i)),
            pl.BlockSpec((1, Eb), lambda i: (0, 0)),
        ],
        out_specs=[
            pl.BlockSpec((LB, 128), lambda i: (i, 0)),
            pl.BlockSpec((LB, B * Ec), lambda i: (i, 0)),
        ],
        out_shape=[
            jax.ShapeDtypeStruct((L1, 128), jnp.float32),
            jax.ShapeDtypeStruct((L1, B * Ec), jnp.float32),
        ],
    )(mat_bT, mat_cT, w_b.reshape(1, Eb))


def _sc_gather(pb_pack, c_pack, idx_b, idx_c, L):
    rows_w = L // _NW
    mesh = plsc.VectorSubcoreMesh(core_axis_name="c", subcore_axis_name="s")

    @functools.partial(
        pl.kernel,
        out_type=(jax.ShapeDtypeStruct((L, 128), jnp.float32),
                  jax.ShapeDtypeStruct((L, 128), jnp.float32)),
        mesh=mesh,
        scratch_types=[
            pltpu.VMEM((_NW, rows_w), jnp.int32),
            pltpu.VMEM((_NW, rows_w), jnp.int32),
            pltpu.VMEM((rows_w, 128), jnp.float32),
            pltpu.VMEM((rows_w, 128), jnp.float32),
            pltpu.SemaphoreType.DMA,
        ],
    )
    def gather_k(pbp_h, cp_h, ib_h, ic_h, pbg_h, cg_h,
                 ib_v, ic_v, rb_v, rc_v, sem):
        wid = lax.axis_index("s") * _NC + lax.axis_index("c")
        base = wid * rows_w
        pltpu.sync_copy(ib_h, ib_v)
        pltpu.sync_copy(ic_h, ic_v)
        cb = pltpu.async_copy(pbp_h.at[ib_v.at[wid]], rb_v, sem)
        cc = pltpu.async_copy(cp_h.at[ic_v.at[wid]], rc_v, sem)
        cb.wait()
        cc.wait()
        pltpu.sync_copy(rb_v, pbg_h.at[pl.ds(base, rows_w)])
        pltpu.sync_copy(rc_v, cg_h.at[pl.ds(base, rows_w)])

    return gather_k(pb_pack, c_pack, idx_b, idx_c)


def _dense_body(pbg_ref, cg_ref, dtT_ref, wc_ref, l1b_ref, a_ref,
                bias_ref, l2_ref, l2b_ref, out_ref):
    B = dtT_ref.shape[1]
    Ec = wc_ref.shape[0]
    for b in range(B):
        c_b = cg_ref[:, b * Ec:(b + 1) * Ec]
        rc = jnp.dot(c_b, wc_ref[...], preferred_element_type=jnp.float32)
        x = jnp.dot(c_b, l1b_ref[...], preferred_element_type=jnp.float32)
        x = x + dtT_ref[:, b:b + 1] * a_ref[...] + bias_ref[...]
        xa = 1.0 / (1.0 + jnp.exp(-x))
        t = jnp.dot(xa, l2_ref[...], preferred_element_type=jnp.float32)
        rate = pbg_ref[:, b:b + 1] + rc + t + l2b_ref[...]
        out_ref[:, b:b + 1] = (jnp.maximum(rate, 0.0)
                               + jnp.log1p(jnp.exp(-jnp.abs(rate))))


def kernel(mat_b, mat_c, arr_b_idx, arr_c_idx, arr_delta_t,
           w_b, w_c, lin1a_w, lin1a_b, lin1b_w, lin1b_b, lin2_w, lin2_b):
    B, L1, Eb = mat_b.shape
    _, L2, Ec = mat_c.shape
    L = arr_b_idx.shape[1]
    rows_w = L // _NW

    ib = arr_b_idx.reshape(_NW, rows_w).astype(jnp.int32)
    ic = arr_c_idx.reshape(_NW, rows_w).astype(jnp.int32)

    pb_pack, c_pack = _tc_pack(jnp.transpose(mat_b, (0, 2, 1)),
                               jnp.transpose(mat_c, (0, 2, 1)), w_b)

    pbg, cg = _sc_gather(pb_pack, c_pack, ib, ic, L)

    dtT = arr_delta_t.astype(jnp.float32).T
    wc_col = w_c.reshape(Ec, 1)
    l1bT = lin1b_w.T
    a_row = lin1a_w.reshape(1, -1)
    bias_row = (lin1a_b + lin1b_b).reshape(1, -1)
    l2_col = lin2_w.reshape(-1, 1)
    l2b = lin2_b.reshape(1, 1)

    H = lin1b_w.shape[0]
    out = pl.pallas_call(
        _dense_body,
        in_specs=[
            pl.BlockSpec((L, 128), lambda: (0, 0)),
            pl.BlockSpec((L, 128), lambda: (0, 0)),
            pl.BlockSpec((L, B), lambda: (0, 0)),
            pl.BlockSpec((Ec, 1), lambda: (0, 0)),
            pl.BlockSpec((Ec, H), lambda: (0, 0)),
            pl.BlockSpec((1, H), lambda: (0, 0)),
            pl.BlockSpec((1, H), lambda: (0, 0)),
            pl.BlockSpec((H, 1), lambda: (0, 0)),
            pl.BlockSpec((1, 1), lambda: (0, 0)),
        ],
        out_specs=pl.BlockSpec((L, B), lambda: (0, 0)),
        out_shape=jax.ShapeDtypeStruct((L, B), jnp.float32),
    )(pbg, cg, dtT, wc_col, l1bT, a_row, bias_row, l2_col, l2b)

    return out.T


# speedup vs baseline: 1.0208x; 1.0035x over previous
import functools

import jax
import jax.numpy as jnp
from jax import lax
from jax.experimental import pallas as pl
from jax.experimental.pallas import tpu as pltpu
from jax.experimental.pallas import tpu_sc as plsc

_NC, _NS = 2, 16
_NW = _NC * _NS


def _pack_body(tbT_ref, tcT_ref, wb_ref, pbp_ref, cp_ref):
    B = tbT_ref.shape[0]
    Ec = tcT_ref.shape[1]
    LB = tbT_ref.shape[2]
    eye = (lax.broadcasted_iota(jnp.int32, (Ec, Ec), 0)
           == lax.broadcasted_iota(jnp.int32, (Ec, Ec), 1)).astype(jnp.float32)
    del LB, eye
    for b in range(B):
        cp_ref[:, b * Ec:(b + 1) * Ec] = tcT_ref[b].T
        pbp_ref[:, b:b + 1] = jnp.dot(
            wb_ref[...], tbT_ref[b], preferred_element_type=jnp.float32).T


def _tc_pack(mat_bT, mat_cT, w_b):
    B, Eb, L1 = mat_bT.shape
    Ec = mat_cT.shape[1]
    LB = 2048
    return pl.pallas_call(
        _pack_body,
        grid=(pl.cdiv(L1, LB),),
        in_specs=[
            pl.BlockSpec((B, Eb, LB), lambda i: (0, 0, i)),
            pl.BlockSpec((B, Ec, LB), lambda i: (0, 0, i)),
            pl.BlockSpec((1, Eb), lambda i: (0, 0)),
        ],
        out_specs=[
            pl.BlockSpec((LB, 128), lambda i: (i, 0)),
            pl.BlockSpec((LB, B * Ec), lambda i: (i, 0)),
        ],
        out_shape=[
            jax.ShapeDtypeStruct((L1, 128), jnp.float32),
            jax.ShapeDtypeStruct((L1, B * Ec), jnp.float32),
        ],
    )(mat_bT, mat_cT, w_b.reshape(1, Eb))


def _sc_gather(pb_pack, c_pack, idx_b, idx_c, L):
    rows_w = L // _NW
    mesh = plsc.VectorSubcoreMesh(core_axis_name="c", subcore_axis_name="s")

    @functools.partial(
        pl.kernel,
        out_type=(jax.ShapeDtypeStruct((L, 128), jnp.float32),
                  jax.ShapeDtypeStruct((L, 128), jnp.float32)),
        mesh=mesh,
        scratch_types=[
            pltpu.VMEM((_NW, rows_w), jnp.int32),
            pltpu.VMEM((_NW, rows_w), jnp.int32),
            pltpu.VMEM((rows_w, 128), jnp.float32),
            pltpu.VMEM((rows_w, 128), jnp.float32),
            pltpu.SemaphoreType.DMA,
        ],
    )
    def gather_k(pbp_h, cp_h, ib_h, ic_h, pbg_h, cg_h,
                 ib_v, ic_v, rb_v, rc_v, sem):
        wid = lax.axis_index("s") * _NC + lax.axis_index("c")
        base = wid * rows_w
        pltpu.sync_copy(ib_h, ib_v)
        pltpu.sync_copy(ic_h, ic_v)
        cb = pltpu.async_copy(pbp_h.at[ib_v.at[wid]], rb_v, sem)
        cc = pltpu.async_copy(cp_h.at[ic_v.at[wid]], rc_v, sem)
        cb.wait()
        cc.wait()
        pltpu.sync_copy(rb_v, pbg_h.at[pl.ds(base, rows_w)])
        pltpu.sync_copy(rc_v, cg_h.at[pl.ds(base, rows_w)])

    return gather_k(pb_pack, c_pack, idx_b, idx_c)


def _dense_body(pbg_ref, cg_ref, dtT_ref, wc_ref, l1b_ref, a_ref,
                bias_ref, l2_ref, l2b_ref, out_ref):
    B = dtT_ref.shape[1]
    Ec = wc_ref.shape[0]
    for b in range(B):
        c_b = cg_ref[:, b * Ec:(b + 1) * Ec]
        rc = jnp.dot(c_b, wc_ref[...], preferred_element_type=jnp.float32)
        x = jnp.dot(c_b, l1b_ref[...], preferred_element_type=jnp.float32)
        x = x + dtT_ref[:, b:b + 1] * a_ref[...] + bias_ref[...]
        xa = 1.0 / (1.0 + jnp.exp(-x))
        t = jnp.dot(xa, l2_ref[...], preferred_element_type=jnp.float32)
        rate = pbg_ref[:, b:b + 1] + rc + t + l2b_ref[...]
        out_ref[:, b:b + 1] = (jnp.maximum(rate, 0.0)
                               + jnp.log1p(jnp.exp(-jnp.abs(rate))))


def kernel(mat_b, mat_c, arr_b_idx, arr_c_idx, arr_delta_t,
           w_b, w_c, lin1a_w, lin1a_b, lin1b_w, lin1b_b, lin2_w, lin2_b):
    B, L1, Eb = mat_b.shape
    _, L2, Ec = mat_c.shape
    L = arr_b_idx.shape[1]
    rows_w = L // _NW

    ib = arr_b_idx.reshape(_NW, rows_w).astype(jnp.int32)
    ic = arr_c_idx.reshape(_NW, rows_w).astype(jnp.int32)

    pb_pack, c_pack = _tc_pack(jnp.transpose(mat_b, (0, 2, 1)),
                               jnp.transpose(mat_c, (0, 2, 1)), w_b)

    pbg, cg = _sc_gather(pb_pack, c_pack, ib, ic, L)

    dtT = arr_delta_t.astype(jnp.float32).T
    wc_col = w_c.reshape(Ec, 1)
    l1bT = lin1b_w.T
    a_row = lin1a_w.reshape(1, -1)
    bias_row = (lin1a_b + lin1b_b).reshape(1, -1)
    l2_col = lin2_w.reshape(-1, 1)
    l2b = lin2_b.reshape(1, 1)

    H = lin1b_w.shape[0]
    out = pl.pallas_call(
        _dense_body,
        in_specs=[
            pl.BlockSpec((L, 128), lambda: (0, 0)),
            pl.BlockSpec((L, 128), lambda: (0, 0)),
            pl.BlockSpec((L, B), lambda: (0, 0)),
            pl.BlockSpec((Ec, 1), lambda: (0, 0)),
            pl.BlockSpec((Ec, H), lambda: (0, 0)),
            pl.BlockSpec((1, H), lambda: (0, 0)),
            pl.BlockSpec((1, H), lambda: (0, 0)),
            pl.BlockSpec((H, 1), lambda: (0, 0)),
            pl.BlockSpec((1, 1), lambda: (0, 0)),
        ],
        out_specs=pl.BlockSpec((L, B), lambda: (0, 0)),
        out_shape=jax.ShapeDtypeStruct((L, B), jnp.float32),
    )(pbg, cg, dtT, wc_col, l1bT, a_row, bias_row, l2_col, l2b)

    return out.T


# speedup vs baseline: 1.5168x; 1.4859x over previous
import functools

import jax
import jax.numpy as jnp
from jax import lax
from jax.experimental import pallas as pl
from jax.experimental.pallas import tpu as pltpu
from jax.experimental.pallas import tpu_sc as plsc

_NC, _NS = 2, 16
_NW = _NC * _NS
_CH = 128


def _sc_gather(flat_b, flat_c, gq_b, gq_c, rows, eb, ec):
    rows_w = rows // _NW
    n_ch = rows_w // _CH
    mesh = plsc.VectorSubcoreMesh(core_axis_name="c", subcore_axis_name="s")

    @functools.partial(
        pl.kernel,
        out_type=(jax.ShapeDtypeStruct((rows, eb), jnp.float32),
                  jax.ShapeDtypeStruct((rows, ec), jnp.float32)),
        mesh=mesh,
        scratch_types=[
            pltpu.VMEM((n_ch, _CH), jnp.int32),
            pltpu.VMEM((n_ch, _CH), jnp.int32),
            pltpu.VMEM((_CH, eb), jnp.float32),
            pltpu.VMEM((_CH, ec), jnp.float32),
            pltpu.SemaphoreType.DMA,
        ],
    )
    def gather_k(tb_h, tc_h, gqb_h, gqc_h, bg_h, cg_h,
                 gqb_v, gqc_v, stgb, stgc, sem):
        wid = lax.axis_index("s") * _NC + lax.axis_index("c")
        base = wid * rows_w
        pltpu.sync_copy(gqb_h.at[wid], gqb_v)
        pltpu.sync_copy(gqc_h.at[wid], gqc_v)

        @pl.loop(0, n_ch)
        def chunk(ch):
            cps = []
            for g in range(_CH // 16):
                qbv = gqb_v[ch, pl.ds(g * 16, 16)]
                qcv = gqc_v[ch, pl.ds(g * 16, 16)]
                for j in range(16):
                    k = g * 16 + j
                    cps.append(pltpu.async_copy(
                        tb_h.at[qbv[j]], stgb.at[k], sem))
                    cps.append(pltpu.async_copy(
                        tc_h.at[qcv[j]], stgc.at[k], sem))
            for cp in cps:
                cp.wait()
            pltpu.sync_copy(stgb, bg_h.at[pl.ds(base + ch * _CH, _CH)])
            pltpu.sync_copy(stgc, cg_h.at[pl.ds(base + ch * _CH, _CH)])

    return gather_k(flat_b, flat_c, gq_b, gq_c)


def _tc_body(bg_ref, cg_ref, dt_ref, wb_ref, wc_ref, l1b_ref, a_ref,
             bias_ref, l2_ref, l2b_ref, out_ref):
    bg = bg_ref[...]
    cg = cg_ref[...]
    dt = dt_ref[...]
    rb = jnp.dot(bg, wb_ref[...], preferred_element_type=jnp.float32)
    rc = jnp.dot(cg, wc_ref[...], preferred_element_type=jnp.float32)
    x = jnp.dot(cg, l1b_ref[...], preferred_element_type=jnp.float32)
    x = x + dt * a_ref[...] + bias_ref[...]
    xa = 1.0 / (1.0 + jnp.exp(-x))
    t = jnp.dot(xa, l2_ref[...], preferred_element_type=jnp.float32)
    rate = rb + rc + t + l2b_ref[...]
    out_ref[...] = jnp.maximum(rate, 0.0) + jnp.log1p(jnp.exp(-jnp.abs(rate)))


def kernel(mat_b, mat_c, arr_b_idx, arr_c_idx, arr_delta_t,
           w_b, w_c, lin1a_w, lin1a_b, lin1b_w, lin1b_b, lin2_w, lin2_b):
    B, L1, Eb = mat_b.shape
    _, L2, Ec = mat_c.shape
    L = arr_b_idx.shape[1]
    R = B * L
    rows_w = R // _NW
    n_ch = rows_w // _CH

    ib = arr_b_idx.reshape(-1).astype(jnp.int32)
    ic = arr_c_idx.reshape(-1).astype(jnp.int32)
    boff = jnp.arange(B, dtype=jnp.int32)[:, None]
    gq_b = (boff * L1 + ib[None, :]).reshape(_NW, n_ch, _CH)
    gq_c = (boff * L2 + ic[None, :]).reshape(_NW, n_ch, _CH)

    bg, cg = _sc_gather(mat_b.reshape(B * L1, Eb),
                        mat_c.reshape(B * L2, Ec),
                        gq_b, gq_c, R, Eb, Ec)

    dt_col = arr_delta_t.astype(jnp.float32).reshape(R, 1)
    wb_col = w_b.reshape(Eb, 1)
    wc_col = w_c.reshape(Ec, 1)
    l1bT = lin1b_w.T
    a_row = lin1a_w.reshape(1, -1)
    bias_row = (lin1a_b + lin1b_b).reshape(1, -1)
    l2_col = lin2_w.reshape(-1, 1)
    l2b = lin2_b.reshape(1, 1)

    RB = 4096
    grid = R // RB
    H = lin1b_w.shape[0]
    out = pl.pallas_call(
        _tc_body,
        grid=(grid,),
        in_specs=[
            pl.BlockSpec((RB, Eb), lambda i: (i, 0)),
            pl.BlockSpec((RB, Ec), lambda i: (i, 0)),
            pl.BlockSpec((RB, 1), lambda i: (i, 0)),
            pl.BlockSpec((Eb, 1), lambda i: (0, 0)),
            pl.BlockSpec((Ec, 1), lambda i: (0, 0)),
            pl.BlockSpec((Ec, H), lambda i: (0, 0)),
            pl.BlockSpec((1, H), lambda i: (0, 0)),
            pl.BlockSpec((1, H), lambda i: (0, 0)),
            pl.BlockSpec((H, 1), lambda i: (0, 0)),
            pl.BlockSpec((1, 1), lambda i: (0, 0)),
        ],
        out_specs=pl.BlockSpec((RB, 1), lambda i: (i, 0)),
        out_shape=jax.ShapeDtypeStruct((R, 1), jnp.float32),
    )(bg, cg, dt_col, wb_col, wc_col, l1bT, a_row, bias_row, l2_col, l2b)

    return out.reshape(B, L)
